# baseline (speedup 1.0000x reference)
import functools

import jax
import jax.numpy as jnp
from jax import lax
from jax.experimental import pallas as pl
from jax.experimental.pallas import tpu as pltpu
from jax.experimental.pallas import tpu_sc as plsc

DIM = 1024
HIDDEN = 2048
E = 8
EPS = 1e-6
NT = 2048
NS = 2 * NT
BR = 128
NBLK = NS // BR + E
NPAD = NBLK * BR
CH = 512
TH = 512
HT = HIDDEN // TH


def _routing_body(x_ref, ns_ref, rw_ref, xn_ref, dest_ref, be_ref, nv_ref, w0_ref, w1_ref):
    x = x_ref[...]
    ms = jnp.mean(x * x, axis=1, keepdims=True)
    xn = x * (ns_ref[...] * lax.rsqrt(ms + EPS))
    xn_ref[...] = xn
    s = lax.dot_general(xn, rw_ref[...], (((1,), (1,)), ((), ())),
                        preferred_element_type=jnp.float32)
    iota8 = lax.broadcasted_iota(jnp.int32, (NT, E), 1)
    m0 = jnp.max(s, axis=1, keepdims=True)
    i0 = jnp.min(jnp.where(s == m0, iota8, E), axis=1, keepdims=True)
    s2 = jnp.where(iota8 == i0, -jnp.inf, s)
    m1 = jnp.max(s2, axis=1, keepdims=True)
    i1 = jnp.min(jnp.where(s2 == m1, iota8, E), axis=1, keepdims=True)
    w0_ref[...] = 1.0 / (1.0 + jnp.exp(m1 - m0))
    w1_ref[...] = 1.0 / (1.0 + jnp.exp(m0 - m1))
    oh0 = (iota8 == i0).astype(jnp.float32)
    oh1 = (iota8 == i1).astype(jnp.float32)
    counts = (jnp.sum(oh0, axis=0, keepdims=True)
              + jnp.sum(oh1, axis=0, keepdims=True))
    nb = jnp.floor((counts + (BR - 1)) * (1.0 / BR))
    ii = lax.broadcasted_iota(jnp.int32, (E, E), 0)
    jj = lax.broadcasted_iota(jnp.int32, (E, E), 1)
    tl = (ii <= jj).astype(jnp.float32)
    cum = lax.dot_general(nb, tl, (((1,), (0,)), ((), ())),
                          preferred_element_type=jnp.float32)
    off = BR * (cum - nb)
    bb = lax.broadcasted_iota(jnp.int32, (NBLK, E), 0).astype(jnp.float32)
    be_ref[...] = jnp.minimum(
        jnp.sum((bb >= cum).astype(jnp.int32), axis=1, keepdims=True), E - 1)
    nv_ref[...] = cum[:, E - 1:E].astype(jnp.int32)
    ci = lax.broadcasted_iota(jnp.int32, (CH, CH), 0)
    cj = lax.broadcasted_iota(jnp.int32, (CH, CH), 1)
    tri = (ci >= cj).astype(jnp.float32)
    m_all = jnp.concatenate([oh0, oh1], axis=0)
    carry = jnp.zeros((1, E), jnp.float32)
    for c in range(NS // CH):
        mc = m_all[c * CH:(c + 1) * CH, :]
        r = lax.dot_general(tri, mc, (((1,), (0,)), ((), ())),
                            preferred_element_type=jnp.float32) + carry
        carry = carry + jnp.sum(mc, axis=0, keepdims=True)
        dest_c = jnp.sum(mc * (off + r - 1.0), axis=1, keepdims=True)
        dest_ref[c * CH:(c + 1) * CH, :] = dest_c.astype(jnp.int32)


def _routing(x2, norm_scale, router_w):
    return pl.pallas_call(
        _routing_body,
        out_shape=(
            jax.ShapeDtypeStruct((NT, DIM), jnp.float32),
            jax.ShapeDtypeStruct((NS, 1), jnp.int32),
            jax.ShapeDtypeStruct((NBLK, 1), jnp.int32),
            jax.ShapeDtypeStruct((1, 1), jnp.int32),
            jax.ShapeDtypeStruct((NT, 1), jnp.float32),
            jax.ShapeDtypeStruct((NT, 1), jnp.float32),
        ),
    )(x2, norm_scale.reshape(1, DIM), router_w)


_NC = 2
_NW = 32
_RXS = NPAD // _NW
_RDG = NS // _NW
_HW = DIM // 2


def _sc_mesh():
    return plsc.VectorSubcoreMesh(core_axis_name="c", subcore_axis_name="s")


def _sc_scatter_xs(dest, xn):
    @functools.partial(
        pl.kernel,
        mesh=_sc_mesh(),
        out_type=jax.ShapeDtypeStruct((NPAD, DIM), jnp.float32),
        scratch_types=[
            pltpu.VMEM((64,), jnp.int32),
            pltpu.VMEM((64, DIM), jnp.float32),
            pltpu.SemaphoreType.DMA,
        ],
    )
    def k(dest_hbm, xn_hbm, xs_hbm, idx_v, rows_v, sem):
        wid = lax.axis_index("s") * _NC + lax.axis_index("c")
        base = pl.multiple_of(wid * _RDG, _RDG)
        tok = pl.multiple_of(base & (NT - 1), _RDG)
        for c in range(2):
            pltpu.sync_copy(dest_hbm.at[pl.ds(base + c * 64, 64)], idx_v)
            pltpu.sync_copy(xn_hbm.at[pl.ds(tok + c * 64, 64)], rows_v)
            pltpu.async_copy(rows_v, xs_hbm.at[idx_v], sem).wait()

    return k(dest, xn)


def _sc_gather_comb(dest, d_sorted):
    @functools.partial(
        pl.kernel,
        mesh=_sc_mesh(),
        out_type=jax.ShapeDtypeStruct((NS, DIM), jnp.float32),
        scratch_types=[
            pltpu.VMEM((64,), jnp.int32),
            pltpu.VMEM((64, DIM), jnp.float32),
            pltpu.SemaphoreType.DMA,
        ],
    )
    def k(dest_hbm, d_hbm, dg_hbm, idx_v, rows_v, sem):
        wid = lax.axis_index("s") * _NC + lax.axis_index("c")
        base = pl.multiple_of(wid * _RDG, _RDG)
        for c in range(2):
            pltpu.sync_copy(dest_hbm.at[pl.ds(base + c * 64, 64)], idx_v)
            pltpu.async_copy(d_hbm.at[idx_v], rows_v, sem).wait()
            pltpu.sync_copy(rows_v, dg_hbm.at[pl.ds(base + c * 64, 64)])

    return k(dest, d_sorted)


def _gmm_body(be_ref, nv_ref, xs_ref, uw_ref, gw_ref, dw_ref, out_ref, acc_ref):
    j = pl.program_id(0)
    i = pl.program_id(1)
    valid = i < nv_ref[0]

    @pl.when(valid)
    def _():
        _gmm_block(xs_ref, uw_ref, gw_ref, dw_ref, out_ref, acc_ref, i, j)


def _gmm_block(xs_ref, uw_ref, gw_ref, dw_ref, out_ref, acc_ref, i, j):
    row = pl.ds(pl.multiple_of(i * BR, BR), BR)
    xb = xs_ref[row, :].astype(jnp.bfloat16)
    u = lax.dot_general(xb, uw_ref[0], (((1,), (1,)), ((), ())),
                        preferred_element_type=jnp.float32)
    g = lax.dot_general(xb, gw_ref[0], (((1,), (1,)), ((), ())),
                        preferred_element_type=jnp.float32)
    ub = u.astype(jnp.bfloat16)
    gf = g.astype(jnp.bfloat16).astype(jnp.float32)
    sil = (gf / (1.0 + jnp.exp(-gf))).astype(jnp.bfloat16)
    h = ub * sil
    d = lax.dot_general(h, dw_ref[0], (((1,), (1,)), ((), ())),
                        preferred_element_type=jnp.float32)

    @pl.when(j == 0)
    def _():
        acc_ref[row, :] = d

    @pl.when(j > 0)
    def _():
        acc_ref[row, :] += d

    @pl.when(j == HT - 1)
    def _():
        out_ref[...] = acc_ref[row, :].astype(jnp.bfloat16).astype(jnp.float32)


def _gmm(be, nv, xs_bf, up_bf, down_bf):
    grid_spec = pltpu.PrefetchScalarGridSpec(
        num_scalar_prefetch=2,
        grid=(HT, NBLK),
        in_specs=[
            pl.BlockSpec((NPAD, DIM), lambda j, i, be, nv: (0, 0)),
            pl.BlockSpec((1, TH, DIM), lambda j, i, be, nv: (be[i], j, 0)),
            pl.BlockSpec((1, TH, DIM), lambda j, i, be, nv: (be[i], j + HT, 0)),
            pl.BlockSpec((1, DIM, TH), lambda j, i, be, nv: (be[i], 0, j)),
        ],
        out_specs=pl.BlockSpec((BR, DIM), lambda j, i, be, nv: (i, 0)),
        scratch_shapes=[pltpu.VMEM((NPAD, DIM), jnp.float32)],
    )
    return pl.pallas_call(
        _gmm_body,
        grid_spec=grid_spec,
        out_shape=jax.ShapeDtypeStruct((NPAD, DIM), jnp.float32),
    )(be, nv, xs_bf, up_bf, up_bf, down_bf)


def _combine_body(d0_ref, d1_ref, w0_ref, w1_ref, x_ref, o_ref):
    o_ref[...] = (w0_ref[...] * d0_ref[...] + w1_ref[...] * d1_ref[...]
                  + x_ref[...])


def _combine(d0, d1, w0, w1, x2):
    nb = NT // BR
    return pl.pallas_call(
        _combine_body,
        grid=(nb,),
        in_specs=[
            pl.BlockSpec((BR, DIM), lambda i: (i, 0)),
            pl.BlockSpec((BR, DIM), lambda i: (i, 0)),
            pl.BlockSpec((BR, 1), lambda i: (i, 0)),
            pl.BlockSpec((BR, 1), lambda i: (i, 0)),
            pl.BlockSpec((BR, DIM), lambda i: (i, 0)),
        ],
        out_specs=pl.BlockSpec((BR, DIM), lambda i: (i, 0)),
        out_shape=jax.ShapeDtypeStruct((NT, DIM), jnp.float32),
    )(d0, d1, w0, w1, x2)


def kernel(x, norm_scale, router_w, up_w, down_w):
    x2 = x.reshape(NT, DIM)
    xn_bf, dest2, be2, nv2, w0, w1 = _routing(x2, norm_scale, router_w)
    dest = dest2.reshape(NS)
    be = be2.reshape(NBLK)
    nv = nv2.reshape(1)
    up_bf = up_w.astype(jnp.bfloat16)
    down_bf = down_w.astype(jnp.bfloat16)
    xs_bf = _sc_scatter_xs(dest, xn_bf)
    d_sorted = _gmm(be, nv, xs_bf, up_bf, down_bf)
    dg = _sc_gather_comb(dest, d_sorted)
    out = _combine(dg[:NT], dg[NT:], w0, w1, x2)
    return out.reshape(x.shape)


# speedup vs baseline: 1.5681x; 1.5681x over previous
import functools

import jax
import jax.numpy as jnp
from jax import lax
from jax.experimental import pallas as pl
from jax.experimental.pallas import tpu as pltpu
from jax.experimental.pallas import tpu_sc as plsc

DIM = 1024
HIDDEN = 2048
E = 8
EPS = 1e-6
NT = 2048
NS = 2 * NT
BR = 256
NBLK = NS // BR + E
NPAD = NBLK * BR
CH = 512
TH = 512
HT = HIDDEN // TH


def _routing_body(x_ref, ns_ref, rw_ref, xn_ref, dest_ref, be_ref, nv_ref, w0_ref, w1_ref):
    x = x_ref[...]
    ms = jnp.mean(x * x, axis=1, keepdims=True)
    xn = x * (ns_ref[...] * lax.rsqrt(ms + EPS))
    xn_ref[...] = xn
    s = lax.dot_general(xn, rw_ref[...], (((1,), (1,)), ((), ())),
                        preferred_element_type=jnp.float32)
    iota8 = lax.broadcasted_iota(jnp.int32, (NT, E), 1)
    m0 = jnp.max(s, axis=1, keepdims=True)
    i0 = jnp.min(jnp.where(s == m0, iota8, E), axis=1, keepdims=True)
    s2 = jnp.where(iota8 == i0, -jnp.inf, s)
    m1 = jnp.max(s2, axis=1, keepdims=True)
    i1 = jnp.min(jnp.where(s2 == m1, iota8, E), axis=1, keepdims=True)
    w0_ref[...] = 1.0 / (1.0 + jnp.exp(m1 - m0))
    w1_ref[...] = 1.0 / (1.0 + jnp.exp(m0 - m1))
    oh0 = (iota8 == i0).astype(jnp.float32)
    oh1 = (iota8 == i1).astype(jnp.float32)
    counts = (jnp.sum(oh0, axis=0, keepdims=True)
              + jnp.sum(oh1, axis=0, keepdims=True))
    nb = jnp.floor((counts + (BR - 1)) * (1.0 / BR))
    ii = lax.broadcasted_iota(jnp.int32, (E, E), 0)
    jj = lax.broadcasted_iota(jnp.int32, (E, E), 1)
    tl = (ii <= jj).astype(jnp.float32)
    cum = lax.dot_general(nb, tl, (((1,), (0,)), ((), ())),
                          preferred_element_type=jnp.float32)
    off = BR * (cum - nb)
    bb = lax.broadcasted_iota(jnp.int32, (NBLK, E), 0).astype(jnp.float32)
    be_ref[...] = jnp.minimum(
        jnp.sum((bb >= cum).astype(jnp.int32), axis=1, keepdims=True), E - 1)
    nv_ref[...] = cum[:, E - 1:E].astype(jnp.int32)
    ci = lax.broadcasted_iota(jnp.int32, (CH, CH), 0)
    cj = lax.broadcasted_iota(jnp.int32, (CH, CH), 1)
    tri = (ci >= cj).astype(jnp.float32)
    m_all = jnp.concatenate([oh0, oh1], axis=0)
    carry = jnp.zeros((1, E), jnp.float32)
    for c in range(NS // CH):
        mc = m_all[c * CH:(c + 1) * CH, :]
        r = lax.dot_general(tri, mc, (((1,), (0,)), ((), ())),
                            preferred_element_type=jnp.float32) + carry
        carry = carry + jnp.sum(mc, axis=0, keepdims=True)
        dest_c = jnp.sum(mc * (off + r - 1.0), axis=1, keepdims=True)
        dest_ref[c * CH:(c + 1) * CH, :] = dest_c.astype(jnp.int32)


def _routing(x2, norm_scale, router_w):
    return pl.pallas_call(
        _routing_body,
        out_shape=(
            jax.ShapeDtypeStruct((NT, DIM), jnp.float32),
            jax.ShapeDtypeStruct((NS, 1), jnp.int32),
            jax.ShapeDtypeStruct((NBLK, 1), jnp.int32),
            jax.ShapeDtypeStruct((1, 1), jnp.int32),
            jax.ShapeDtypeStruct((NT, 1), jnp.float32),
            jax.ShapeDtypeStruct((NT, 1), jnp.float32),
        ),
    )(x2, norm_scale.reshape(1, DIM), router_w)


_NC = 2
_NW = 32
_RXS = NPAD // _NW
_RDG = NS // _NW
_HW = DIM // 2


def _sc_mesh():
    return plsc.VectorSubcoreMesh(core_axis_name="c", subcore_axis_name="s")


def _sc_scatter_xs(dest, xn):
    @functools.partial(
        pl.kernel,
        mesh=_sc_mesh(),
        out_type=jax.ShapeDtypeStruct((NPAD, DIM), jnp.float32),
        scratch_types=[
            pltpu.VMEM((64,), jnp.int32),
            pltpu.VMEM((64, DIM), jnp.float32),
            pltpu.SemaphoreType.DMA,
        ],
    )
    def k(dest_hbm, xn_hbm, xs_hbm, idx_v, rows_v, sem):
        wid = lax.axis_index("s") * _NC + lax.axis_index("c")
        base = pl.multiple_of(wid * _RDG, _RDG)
        tok = pl.multiple_of(base & (NT - 1), _RDG)
        for c in range(2):
            pltpu.sync_copy(dest_hbm.at[pl.ds(base + c * 64, 64)], idx_v)
            pltpu.sync_copy(xn_hbm.at[pl.ds(tok + c * 64, 64)], rows_v)
            pltpu.async_copy(rows_v, xs_hbm.at[idx_v], sem).wait()

    return k(dest, xn)


def _sc_gather_comb(dest, d_sorted):
    @functools.partial(
        pl.kernel,
        mesh=_sc_mesh(),
        out_type=jax.ShapeDtypeStruct((NS, DIM), jnp.float32),
        scratch_types=[
            pltpu.VMEM((64,), jnp.int32),
            pltpu.VMEM((64, DIM), jnp.float32),
            pltpu.SemaphoreType.DMA,
        ],
    )
    def k(dest_hbm, d_hbm, dg_hbm, idx_v, rows_v, sem):
        wid = lax.axis_index("s") * _NC + lax.axis_index("c")
        base = pl.multiple_of(wid * _RDG, _RDG)
        for c in range(2):
            pltpu.sync_copy(dest_hbm.at[pl.ds(base + c * 64, 64)], idx_v)
            pltpu.async_copy(d_hbm.at[idx_v], rows_v, sem).wait()
            pltpu.sync_copy(rows_v, dg_hbm.at[pl.ds(base + c * 64, 64)])

    return k(dest, d_sorted)


def _gmm_body(be_ref, nv_ref, xs_ref, uw_ref, gw_ref, dw_ref, out_ref):
    valid = pl.program_id(0) < nv_ref[0]

    @pl.when(valid)
    def _():
        _gmm_block(xs_ref, uw_ref, gw_ref, dw_ref, out_ref)


def _gmm_block(xs_ref, uw_ref, gw_ref, dw_ref, out_ref):
    xb = xs_ref[...].astype(jnp.bfloat16)
    u = lax.dot_general(xb, uw_ref[0, 0], (((1,), (1,)), ((), ())),
                        preferred_element_type=jnp.float32)
    g = lax.dot_general(xb, gw_ref[0, 0], (((1,), (1,)), ((), ())),
                        preferred_element_type=jnp.float32)
    ub = u.astype(jnp.bfloat16)
    gf = g.astype(jnp.bfloat16).astype(jnp.float32)
    sil = (gf / (1.0 + jnp.exp(-gf))).astype(jnp.bfloat16)
    h = ub * sil
    d = lax.dot_general(h, dw_ref[0], (((1,), (1,)), ((), ())),
                        preferred_element_type=jnp.float32)
    out_ref[...] = d.astype(jnp.bfloat16).astype(jnp.float32)


def _gmm(be, nv, xs_bf, up_bf, down_bf):
    grid_spec = pltpu.PrefetchScalarGridSpec(
        num_scalar_prefetch=2,
        grid=(NBLK,),
        in_specs=[
            pl.BlockSpec((BR, DIM), lambda i, be, nv: (i, 0)),
            pl.BlockSpec((1, 1, HIDDEN, DIM), lambda i, be, nv: (be[i], 0, 0, 0)),
            pl.BlockSpec((1, 1, HIDDEN, DIM), lambda i, be, nv: (be[i], 1, 0, 0)),
            pl.BlockSpec((1, DIM, HIDDEN), lambda i, be, nv: (be[i], 0, 0)),
        ],
        out_specs=pl.BlockSpec((BR, DIM), lambda i, be, nv: (i, 0)),
    )
    f = pl.pallas_call(
        _gmm_body,
        grid_spec=grid_spec,
        out_shape=jax.ShapeDtypeStruct((NPAD, DIM), jnp.float32),
    )
    up4 = up_bf.reshape(E, 2, HIDDEN, DIM)
    return f(be, nv, xs_bf, up4, up4, down_bf)


def _combine_body(d0_ref, d1_ref, w0_ref, w1_ref, x_ref, o_ref):
    o_ref[...] = (w0_ref[...] * d0_ref[...] + w1_ref[...] * d1_ref[...]
                  + x_ref[...])


def _combine(d0, d1, w0, w1, x2):
    nb = NT // BR
    return pl.pallas_call(
        _combine_body,
        grid=(nb,),
        in_specs=[
            pl.BlockSpec((BR, DIM), lambda i: (i, 0)),
            pl.BlockSpec((BR, DIM), lambda i: (i, 0)),
            pl.BlockSpec((BR, 1), lambda i: (i, 0)),
            pl.BlockSpec((BR, 1), lambda i: (i, 0)),
            pl.BlockSpec((BR, DIM), lambda i: (i, 0)),
        ],
        out_specs=pl.BlockSpec((BR, DIM), lambda i: (i, 0)),
        out_shape=jax.ShapeDtypeStruct((NT, DIM), jnp.float32),
    )(d0, d1, w0, w1, x2)


def kernel(x, norm_scale, router_w, up_w, down_w):
    x2 = x.reshape(NT, DIM)
    xn_bf, dest2, be2, nv2, w0, w1 = _routing(x2, norm_scale, router_w)
    dest = dest2.reshape(NS)
    be = be2.reshape(NBLK)
    nv = nv2.reshape(1)
    up_bf = up_w.astype(jnp.bfloat16)
    down_bf = down_w.astype(jnp.bfloat16)
    xs_bf = _sc_scatter_xs(dest, xn_bf)
    d_sorted = _gmm(be, nv, xs_bf, up_bf, down_bf)
    dg = _sc_gather_comb(dest, d_sorted)
    out = _combine(dg[:NT], dg[NT:], w0, w1, x2)
    return out.reshape(x.shape)


# speedup vs baseline: 1.6479x; 1.0509x over previous
import functools

import jax
import jax.numpy as jnp
from jax import lax
from jax.experimental import pallas as pl
from jax.experimental.pallas import tpu as pltpu
from jax.experimental.pallas import tpu_sc as plsc

DIM = 1024
HIDDEN = 2048
E = 8
EPS = 1e-6
NT = 2048
NS = 2 * NT
BR = 256
NBLK = NS // BR + E
NPAD = NBLK * BR
CH = 512
TH = 512
HT = HIDDEN // TH


def _routing_body(x_ref, ns_ref, rw_ref, xn_ref, dest_ref, be_ref, nv_ref, w0_ref, w1_ref):
    x = x_ref[...]
    ms = jnp.mean(x * x, axis=1, keepdims=True)
    xn = x * (ns_ref[...] * lax.rsqrt(ms + EPS))
    xn_ref[...] = xn
    s = lax.dot_general(xn, rw_ref[...], (((1,), (1,)), ((), ())),
                        preferred_element_type=jnp.float32)
    iota8 = lax.broadcasted_iota(jnp.int32, (NT, E), 1)
    m0 = jnp.max(s, axis=1, keepdims=True)
    i0 = jnp.min(jnp.where(s == m0, iota8, E), axis=1, keepdims=True)
    s2 = jnp.where(iota8 == i0, -jnp.inf, s)
    m1 = jnp.max(s2, axis=1, keepdims=True)
    i1 = jnp.min(jnp.where(s2 == m1, iota8, E), axis=1, keepdims=True)
    w0_ref[...] = 1.0 / (1.0 + jnp.exp(m1 - m0))
    w1_ref[...] = 1.0 / (1.0 + jnp.exp(m0 - m1))
    oh0 = (iota8 == i0).astype(jnp.float32)
    oh1 = (iota8 == i1).astype(jnp.float32)
    counts = (jnp.sum(oh0, axis=0, keepdims=True)
              + jnp.sum(oh1, axis=0, keepdims=True))
    nb = jnp.floor((counts + (BR - 1)) * (1.0 / BR))
    ii = lax.broadcasted_iota(jnp.int32, (E, E), 0)
    jj = lax.broadcasted_iota(jnp.int32, (E, E), 1)
    tl = (ii <= jj).astype(jnp.float32)
    cum = lax.dot_general(nb, tl, (((1,), (0,)), ((), ())),
                          preferred_element_type=jnp.float32)
    off = BR * (cum - nb)
    bb = lax.broadcasted_iota(jnp.int32, (NBLK, E), 0).astype(jnp.float32)
    be_ref[...] = jnp.minimum(
        jnp.sum((bb >= cum).astype(jnp.int32), axis=1, keepdims=True), E - 1)
    nv_ref[...] = cum[:, E - 1:E].astype(jnp.int32)
    ci = lax.broadcasted_iota(jnp.int32, (CH, CH), 0)
    cj = lax.broadcasted_iota(jnp.int32, (CH, CH), 1)
    tri = (ci >= cj).astype(jnp.float32)
    m_all = jnp.concatenate([oh0, oh1], axis=0)
    carry = jnp.zeros((1, E), jnp.float32)
    for c in range(NS // CH):
        mc = m_all[c * CH:(c + 1) * CH, :]
        r = lax.dot_general(tri, mc, (((1,), (0,)), ((), ())),
                            preferred_element_type=jnp.float32) + carry
        carry = carry + jnp.sum(mc, axis=0, keepdims=True)
        dest_c = jnp.sum(mc * (off + r - 1.0), axis=1, keepdims=True)
        dest_ref[c * CH:(c + 1) * CH, :] = dest_c.astype(jnp.int32)


def _routing(x2, norm_scale, router_w):
    return pl.pallas_call(
        _routing_body,
        out_shape=(
            jax.ShapeDtypeStruct((NT, DIM), jnp.float32),
            jax.ShapeDtypeStruct((NS, 1), jnp.int32),
            jax.ShapeDtypeStruct((NBLK, 1), jnp.int32),
            jax.ShapeDtypeStruct((1, 1), jnp.int32),
            jax.ShapeDtypeStruct((NT, 1), jnp.float32),
            jax.ShapeDtypeStruct((NT, 1), jnp.float32),
        ),
    )(x2, norm_scale.reshape(1, DIM), router_w)


_NC = 2
_NW = 32
_RXS = NPAD // _NW
_RDG = NS // _NW
_HW = DIM // 2


def _sc_mesh():
    return plsc.VectorSubcoreMesh(core_axis_name="c", subcore_axis_name="s")


def _sc_scatter_xs(dest, xn):
    @functools.partial(
        pl.kernel,
        mesh=_sc_mesh(),
        out_type=jax.ShapeDtypeStruct((NPAD, DIM), jnp.float32),
        scratch_types=[
            pltpu.VMEM((56,), jnp.int32),
            pltpu.VMEM((56,), jnp.int32),
            pltpu.VMEM((16,), jnp.int32),
            pltpu.VMEM((56, DIM), jnp.float32),
            pltpu.VMEM((56, DIM), jnp.float32),
            pltpu.SemaphoreType.DMA,
            pltpu.SemaphoreType.DMA,
            pltpu.SemaphoreType.DMA,
            pltpu.SemaphoreType.DMA,
        ],
    )
    def k(dest_hbm, xn_hbm, xs_hbm, idx0, idx1, idx2, rows0, rows1,
          si0, si1, sr0, sr1):
        wid = lax.axis_index("s") * _NC + lax.axis_index("c")
        base = pl.multiple_of(wid * _RDG, _RDG)
        tok = pl.multiple_of(base & (NT - 1), _RDG)
        ci0 = pltpu.async_copy(dest_hbm.at[pl.ds(base, 56)], idx0, si0)
        ci1 = pltpu.async_copy(dest_hbm.at[pl.ds(base + 56, 56)], idx1, si1)
        ci2 = pltpu.async_copy(dest_hbm.at[pl.ds(base + 112, 16)], idx2, sr1)
        cr0 = pltpu.async_copy(xn_hbm.at[pl.ds(tok, 56)], rows0, sr0)
        cr1 = pltpu.async_copy(xn_hbm.at[pl.ds(tok + 56, 56)], rows1, sr0)
        ci0.wait()
        cr0.wait()
        w0 = pltpu.async_copy(rows0, xs_hbm.at[idx0], si0)
        ci1.wait()
        cr1.wait()
        w1 = pltpu.async_copy(rows1, xs_hbm.at[idx1], si1)
        w0.wait()
        cr2 = pltpu.async_copy(xn_hbm.at[pl.ds(tok + 112, 16)],
                               rows0.at[pl.ds(0, 16)], sr0)
        ci2.wait()
        cr2.wait()
        w2 = pltpu.async_copy(rows0.at[pl.ds(0, 16)], xs_hbm.at[idx2], sr1)
        w1.wait()
        w2.wait()

    return k(dest, xn)


def _sc_gather_comb(dest, d_sorted):
    @functools.partial(
        pl.kernel,
        mesh=_sc_mesh(),
        out_type=jax.ShapeDtypeStruct((NS, DIM), jnp.float32),
        scratch_types=[
            pltpu.VMEM((56,), jnp.int32),
            pltpu.VMEM((56,), jnp.int32),
            pltpu.VMEM((16,), jnp.int32),
            pltpu.VMEM((56, DIM), jnp.float32),
            pltpu.VMEM((56, DIM), jnp.float32),
            pltpu.SemaphoreType.DMA,
            pltpu.SemaphoreType.DMA,
            pltpu.SemaphoreType.DMA,
            pltpu.SemaphoreType.DMA,
        ],
    )
    def k(dest_hbm, d_hbm, dg_hbm, idx0, idx1, idx2, rows0, rows1,
          si0, si1, sr0, sr1):
        wid = lax.axis_index("s") * _NC + lax.axis_index("c")
        base = pl.multiple_of(wid * _RDG, _RDG)
        ci0 = pltpu.async_copy(dest_hbm.at[pl.ds(base, 56)], idx0, si0)
        ci1 = pltpu.async_copy(dest_hbm.at[pl.ds(base + 56, 56)], idx1, si1)
        ci2 = pltpu.async_copy(dest_hbm.at[pl.ds(base + 112, 16)], idx2, sr1)
        ci0.wait()
        g0 = pltpu.async_copy(d_hbm.at[idx0], rows0, sr0)
        ci1.wait()
        g1 = pltpu.async_copy(d_hbm.at[idx1], rows1, sr0)
        g0.wait()
        o0 = pltpu.async_copy(rows0, dg_hbm.at[pl.ds(base, 56)], si0)
        g1.wait()
        o1 = pltpu.async_copy(rows1, dg_hbm.at[pl.ds(base + 56, 56)], si1)
        o0.wait()
        ci2.wait()
        g2 = pltpu.async_copy(d_hbm.at[idx2], rows0.at[pl.ds(0, 16)], sr0)
        g2.wait()
        o2 = pltpu.async_copy(rows0.at[pl.ds(0, 16)],
                              dg_hbm.at[pl.ds(base + 112, 16)], sr1)
        o1.wait()
        o2.wait()

    return k(dest, d_sorted)


def _gmm_body(be_ref, nv_ref, xs_ref, uw_ref, gw_ref, dw_ref, out_ref):
    valid = pl.program_id(0) < nv_ref[0]

    @pl.when(valid)
    def _():
        _gmm_block(xs_ref, uw_ref, gw_ref, dw_ref, out_ref)


def _gmm_block(xs_ref, uw_ref, gw_ref, dw_ref, out_ref):
    xb = xs_ref[...].astype(jnp.bfloat16)
    u = lax.dot_general(xb, uw_ref[0, 0], (((1,), (1,)), ((), ())),
                        preferred_element_type=jnp.float32)
    g = lax.dot_general(xb, gw_ref[0, 0], (((1,), (1,)), ((), ())),
                        preferred_element_type=jnp.float32)
    ub = u.astype(jnp.bfloat16)
    gf = g.astype(jnp.bfloat16).astype(jnp.float32)
    sil = (gf / (1.0 + jnp.exp(-gf))).astype(jnp.bfloat16)
    h = ub * sil
    d = lax.dot_general(h, dw_ref[0], (((1,), (1,)), ((), ())),
                        preferred_element_type=jnp.float32)
    out_ref[...] = d.astype(jnp.bfloat16).astype(jnp.float32)


def _gmm(be, nv, xs_bf, up_bf, down_bf):
    grid_spec = pltpu.PrefetchScalarGridSpec(
        num_scalar_prefetch=2,
        grid=(NBLK,),
        in_specs=[
            pl.BlockSpec((BR, DIM), lambda i, be, nv: (i, 0)),
            pl.BlockSpec((1, 1, HIDDEN, DIM), lambda i, be, nv: (be[i], 0, 0, 0)),
            pl.BlockSpec((1, 1, HIDDEN, DIM), lambda i, be, nv: (be[i], 1, 0, 0)),
            pl.BlockSpec((1, DIM, HIDDEN), lambda i, be, nv: (be[i], 0, 0)),
        ],
        out_specs=pl.BlockSpec((BR, DIM), lambda i, be, nv: (i, 0)),
    )
    f = pl.pallas_call(
        _gmm_body,
        grid_spec=grid_spec,
        out_shape=jax.ShapeDtypeStruct((NPAD, DIM), jnp.float32),
    )
    up4 = up_bf.reshape(E, 2, HIDDEN, DIM)
    return f(be, nv, xs_bf, up4, up4, down_bf)


def _combine_body(d0_ref, d1_ref, w0_ref, w1_ref, x_ref, o_ref):
    o_ref[...] = (w0_ref[...] * d0_ref[...] + w1_ref[...] * d1_ref[...]
                  + x_ref[...])


def _combine(dg, w0, w1, x2):
    nb = NT // BR
    return pl.pallas_call(
        _combine_body,
        grid=(nb,),
        in_specs=[
            pl.BlockSpec((BR, DIM), lambda i: (i, 0)),
            pl.BlockSpec((BR, DIM), lambda i: (i + NT // BR, 0)),
            pl.BlockSpec((BR, 1), lambda i: (i, 0)),
            pl.BlockSpec((BR, 1), lambda i: (i, 0)),
            pl.BlockSpec((BR, DIM), lambda i: (i, 0)),
        ],
        out_specs=pl.BlockSpec((BR, DIM), lambda i: (i, 0)),
        out_shape=jax.ShapeDtypeStruct((NT, DIM), jnp.float32),
    )(dg, dg, w0, w1, x2)


def kernel(x, norm_scale, router_w, up_w, down_w):
    x2 = x.reshape(NT, DIM)
    xn_bf, dest2, be2, nv2, w0, w1 = _routing(x2, norm_scale, router_w)
    dest = dest2.reshape(NS)
    be = be2.reshape(NBLK)
    nv = nv2.reshape(1)
    up_bf = up_w.astype(jnp.bfloat16)
    down_bf = down_w.astype(jnp.bfloat16)
    xs_bf = _sc_scatter_xs(dest, xn_bf)
    d_sorted = _gmm(be, nv, xs_bf, up_bf, down_bf)
    dg = _sc_gather_comb(dest, d_sorted)
    out = _combine(dg, w0, w1, x2)
    return out.reshape(x.shape)


# speedup vs baseline: 1.6500x; 1.0012x over previous
import functools

import jax
import jax.numpy as jnp
from jax import lax
from jax.experimental import pallas as pl
from jax.experimental.pallas import tpu as pltpu
from jax.experimental.pallas import tpu_sc as plsc

DIM = 1024
HIDDEN = 2048
E = 8
EPS = 1e-6
NT = 2048
NS = 2 * NT
BR = 256
NBLK = NS // BR + E
NPAD = NBLK * BR
CH = 512


def _routing_body(x_ref, ns_ref, rw_ref, xn_ref, dest_ref, be_ref, nv_ref, w0_ref, w1_ref):
    x = x_ref[...]
    ms = jnp.mean(x * x, axis=1, keepdims=True)
    xn = x * (ns_ref[...] * lax.rsqrt(ms + EPS))
    xn_ref[...] = xn
    s = lax.dot_general(xn, rw_ref[...], (((1,), (1,)), ((), ())),
                        preferred_element_type=jnp.float32)
    iota8 = lax.broadcasted_iota(jnp.int32, (NT, E), 1)
    m0 = jnp.max(s, axis=1, keepdims=True)
    i0 = jnp.min(jnp.where(s == m0, iota8, E), axis=1, keepdims=True)
    s2 = jnp.where(iota8 == i0, -jnp.inf, s)
    m1 = jnp.max(s2, axis=1, keepdims=True)
    i1 = jnp.min(jnp.where(s2 == m1, iota8, E), axis=1, keepdims=True)
    w0_ref[...] = 1.0 / (1.0 + jnp.exp(m1 - m0))
    w1_ref[...] = 1.0 / (1.0 + jnp.exp(m0 - m1))
    oh0 = (iota8 == i0).astype(jnp.float32)
    oh1 = (iota8 == i1).astype(jnp.float32)
    counts = (jnp.sum(oh0, axis=0, keepdims=True)
              + jnp.sum(oh1, axis=0, keepdims=True))
    nb = jnp.floor((counts + (BR - 1)) * (1.0 / BR))
    ii = lax.broadcasted_iota(jnp.int32, (E, E), 0)
    jj = lax.broadcasted_iota(jnp.int32, (E, E), 1)
    tl = (ii <= jj).astype(jnp.float32)
    cum = lax.dot_general(nb, tl, (((1,), (0,)), ((), ())),
                          preferred_element_type=jnp.float32)
    off = BR * (cum - nb)
    bb = lax.broadcasted_iota(jnp.int32, (NBLK, E), 0).astype(jnp.float32)
    be_ref[...] = jnp.minimum(
        jnp.sum((bb >= cum).astype(jnp.int32), axis=1, keepdims=True), E - 1)
    nv_ref[...] = cum[:, E - 1:E].astype(jnp.int32)
    ci = lax.broadcasted_iota(jnp.int32, (CH, CH), 0)
    cj = lax.broadcasted_iota(jnp.int32, (CH, CH), 1)
    tri = (ci >= cj).astype(jnp.float32)
    m_all = jnp.concatenate([oh0, oh1], axis=0)
    carry = jnp.zeros((1, E), jnp.float32)
    for c in range(NS // CH):
        mc = m_all[c * CH:(c + 1) * CH, :]
        r = lax.dot_general(tri, mc, (((1,), (0,)), ((), ())),
                            preferred_element_type=jnp.float32) + carry
        carry = carry + jnp.sum(mc, axis=0, keepdims=True)
        dest_c = jnp.sum(mc * (off + r - 1.0), axis=1, keepdims=True)
        dest_ref[c * CH:(c + 1) * CH, :] = dest_c.astype(jnp.int32)


def _routing(x2, norm_scale, router_w):
    return pl.pallas_call(
        _routing_body,
        out_shape=(
            jax.ShapeDtypeStruct((NT, DIM), jnp.float32),
            jax.ShapeDtypeStruct((NS, 1), jnp.int32),
            jax.ShapeDtypeStruct((NBLK, 1), jnp.int32),
            jax.ShapeDtypeStruct((1, 1), jnp.int32),
            jax.ShapeDtypeStruct((NT, 1), jnp.float32),
            jax.ShapeDtypeStruct((NT, 1), jnp.float32),
        ),
    )(x2, norm_scale.reshape(1, DIM), router_w)


_NC = 2
_NW = 32
_RXS = NPAD // _NW
_RDG = NS // _NW
_HW = DIM // 2


def _sc_mesh():
    return plsc.VectorSubcoreMesh(core_axis_name="c", subcore_axis_name="s")


def _sc_scatter_xs(dest, xn):
    @functools.partial(
        pl.kernel,
        mesh=_sc_mesh(),
        out_type=jax.ShapeDtypeStruct((NPAD, DIM), jnp.float32),
        scratch_types=[
            pltpu.VMEM((56,), jnp.int32),
            pltpu.VMEM((56,), jnp.int32),
            pltpu.VMEM((16,), jnp.int32),
            pltpu.VMEM((56, DIM), jnp.float32),
            pltpu.VMEM((56, DIM), jnp.float32),
            pltpu.SemaphoreType.DMA,
            pltpu.SemaphoreType.DMA,
            pltpu.SemaphoreType.DMA,
            pltpu.SemaphoreType.DMA,
        ],
    )
    def k(dest_hbm, xn_hbm, xs_hbm, idx0, idx1, idx2, rows0, rows1,
          si0, si1, sr0, sr1):
        wid = lax.axis_index("s") * _NC + lax.axis_index("c")
        base = pl.multiple_of(wid * _RDG, _RDG)
        tok = pl.multiple_of(base & (NT - 1), _RDG)
        ci0 = pltpu.async_copy(dest_hbm.at[pl.ds(base, 56)], idx0, si0)
        ci1 = pltpu.async_copy(dest_hbm.at[pl.ds(base + 56, 56)], idx1, si1)
        ci2 = pltpu.async_copy(dest_hbm.at[pl.ds(base + 112, 16)], idx2, sr1)
        cr0 = pltpu.async_copy(xn_hbm.at[pl.ds(tok, 56)], rows0, sr0)
        cr1 = pltpu.async_copy(xn_hbm.at[pl.ds(tok + 56, 56)], rows1, sr0)
        ci0.wait()
        cr0.wait()
        w0 = pltpu.async_copy(rows0, xs_hbm.at[idx0], si0)
        ci1.wait()
        cr1.wait()
        w1 = pltpu.async_copy(rows1, xs_hbm.at[idx1], si1)
        w0.wait()
        cr2 = pltpu.async_copy(xn_hbm.at[pl.ds(tok + 112, 16)],
                               rows0.at[pl.ds(0, 16)], sr0)
        ci2.wait()
        cr2.wait()
        w2 = pltpu.async_copy(rows0.at[pl.ds(0, 16)], xs_hbm.at[idx2], sr1)
        w1.wait()
        w2.wait()

    return k(dest, xn)


def _sc_gather_comb(dest, d_sorted):
    @functools.partial(
        pl.kernel,
        mesh=_sc_mesh(),
        out_type=jax.ShapeDtypeStruct((NS, DIM), jnp.float32),
        scratch_types=[
            pltpu.VMEM((56,), jnp.int32),
            pltpu.VMEM((56,), jnp.int32),
            pltpu.VMEM((16,), jnp.int32),
            pltpu.VMEM((56, DIM), jnp.float32),
            pltpu.VMEM((56, DIM), jnp.float32),
            pltpu.SemaphoreType.DMA,
            pltpu.SemaphoreType.DMA,
            pltpu.SemaphoreType.DMA,
            pltpu.SemaphoreType.DMA,
        ],
    )
    def k(dest_hbm, d_hbm, dg_hbm, idx0, idx1, idx2, rows0, rows1,
          si0, si1, sr0, sr1):
        wid = lax.axis_index("s") * _NC + lax.axis_index("c")
        base = pl.multiple_of(wid * _RDG, _RDG)
        ci0 = pltpu.async_copy(dest_hbm.at[pl.ds(base, 56)], idx0, si0)
        ci1 = pltpu.async_copy(dest_hbm.at[pl.ds(base + 56, 56)], idx1, si1)
        ci2 = pltpu.async_copy(dest_hbm.at[pl.ds(base + 112, 16)], idx2, sr1)
        ci0.wait()
        g0 = pltpu.async_copy(d_hbm.at[idx0], rows0, sr0)
        ci1.wait()
        g1 = pltpu.async_copy(d_hbm.at[idx1], rows1, sr0)
        g0.wait()
        o0 = pltpu.async_copy(rows0, dg_hbm.at[pl.ds(base, 56)], si0)
        g1.wait()
        o1 = pltpu.async_copy(rows1, dg_hbm.at[pl.ds(base + 56, 56)], si1)
        o0.wait()
        ci2.wait()
        g2 = pltpu.async_copy(d_hbm.at[idx2], rows0.at[pl.ds(0, 16)], sr0)
        g2.wait()
        o2 = pltpu.async_copy(rows0.at[pl.ds(0, 16)],
                              dg_hbm.at[pl.ds(base + 112, 16)], sr1)
        o1.wait()
        o2.wait()

    return k(dest, d_sorted)


def _gmm_body(be_ref, nv_ref, xs_ref, uw_ref, gw_ref, dw_ref, out_ref):
    valid = pl.program_id(0) < nv_ref[0]

    @pl.when(valid)
    def _():
        _gmm_block(xs_ref, uw_ref, gw_ref, dw_ref, out_ref)


def _gmm_block(xs_ref, uw_ref, gw_ref, dw_ref, out_ref):
    xb = xs_ref[...].astype(jnp.bfloat16)
    u = lax.dot_general(xb, uw_ref[0, 0], (((1,), (1,)), ((), ())),
                        preferred_element_type=jnp.float32)
    g = lax.dot_general(xb, gw_ref[0, 0], (((1,), (1,)), ((), ())),
                        preferred_element_type=jnp.float32)
    ub = u.astype(jnp.bfloat16)
    gf = g.astype(jnp.bfloat16).astype(jnp.float32)
    sil = (gf / (1.0 + jnp.exp(-gf))).astype(jnp.bfloat16)
    h = ub * sil
    d = lax.dot_general(h, dw_ref[0], (((1,), (1,)), ((), ())),
                        preferred_element_type=jnp.float32)
    out_ref[...] = d.astype(jnp.bfloat16).astype(jnp.float32)


def _gmm(be, nv, xs_bf, up_bf, down_bf):
    grid_spec = pltpu.PrefetchScalarGridSpec(
        num_scalar_prefetch=2,
        grid=(NBLK,),
        in_specs=[
            pl.BlockSpec((BR, DIM), lambda i, be, nv: (i, 0)),
            pl.BlockSpec((1, 1, HIDDEN, DIM), lambda i, be, nv: (be[i], 0, 0, 0)),
            pl.BlockSpec((1, 1, HIDDEN, DIM), lambda i, be, nv: (be[i], 1, 0, 0)),
            pl.BlockSpec((1, DIM, HIDDEN), lambda i, be, nv: (be[i], 0, 0)),
        ],
        out_specs=pl.BlockSpec((BR, DIM), lambda i, be, nv: (i, 0)),
    )
    f = pl.pallas_call(
        _gmm_body,
        grid_spec=grid_spec,
        out_shape=jax.ShapeDtypeStruct((NPAD, DIM), jnp.float32),
    )
    up4 = up_bf.reshape(E, 2, HIDDEN, DIM)
    return f(be, nv, xs_bf, up4, up4, down_bf)


def _combine_body(d0_ref, d1_ref, w0_ref, w1_ref, x_ref, o_ref):
    o_ref[...] = (w0_ref[...] * d0_ref[...] + w1_ref[...] * d1_ref[...]
                  + x_ref[...])


def _combine(dg, w0, w1, x2):
    nb = NT // BR
    return pl.pallas_call(
        _combine_body,
        grid=(nb,),
        in_specs=[
            pl.BlockSpec((BR, DIM), lambda i: (i, 0)),
            pl.BlockSpec((BR, DIM), lambda i: (i + NT // BR, 0)),
            pl.BlockSpec((BR, 1), lambda i: (i, 0)),
            pl.BlockSpec((BR, 1), lambda i: (i, 0)),
            pl.BlockSpec((BR, DIM), lambda i: (i, 0)),
        ],
        out_specs=pl.BlockSpec((BR, DIM), lambda i: (i, 0)),
        out_shape=jax.ShapeDtypeStruct((NT, DIM), jnp.float32),
    )(dg, dg, w0, w1, x2)


def kernel(x, norm_scale, router_w, up_w, down_w):
    x2 = x.reshape(NT, DIM)
    xn_bf, dest2, be2, nv2, w0, w1 = _routing(x2, norm_scale, router_w)
    dest = dest2.reshape(NS)
    be = be2.reshape(NBLK)
    nv = nv2.reshape(1)
    up_bf = up_w.astype(jnp.bfloat16)
    down_bf = down_w.astype(jnp.bfloat16)
    xs_bf = _sc_scatter_xs(dest, xn_bf)
    d_sorted = _gmm(be, nv, xs_bf, up_bf, down_bf)
    dg = _sc_gather_comb(dest, d_sorted)
    out = _combine(dg, w0, w1, x2)
    return out.reshape(x.shape)


# speedup vs baseline: 1.6845x; 1.0209x over previous
import functools

import jax
import jax.numpy as jnp
from jax import lax
from jax.experimental import pallas as pl
from jax.experimental.pallas import tpu as pltpu
from jax.experimental.pallas import tpu_sc as plsc

DIM = 1024
HIDDEN = 2048
E = 8
EPS = 1e-6
NT = 2048
NS = 2 * NT
BR = 512
NBLK = NS // BR + E
NPAD = NBLK * BR
CH = 512


def _routing_body(x_ref, ns_ref, rw_ref, xn_ref, dest_ref, be_ref, nv_ref, w0_ref, w1_ref):
    x = x_ref[...]
    ms = jnp.mean(x * x, axis=1, keepdims=True)
    xn = x * (ns_ref[...] * lax.rsqrt(ms + EPS))
    xn_ref[...] = xn
    s = lax.dot_general(xn, rw_ref[...], (((1,), (1,)), ((), ())),
                        preferred_element_type=jnp.float32)
    iota8 = lax.broadcasted_iota(jnp.int32, (NT, E), 1)
    m0 = jnp.max(s, axis=1, keepdims=True)
    i0 = jnp.min(jnp.where(s == m0, iota8, E), axis=1, keepdims=True)
    s2 = jnp.where(iota8 == i0, -jnp.inf, s)
    m1 = jnp.max(s2, axis=1, keepdims=True)
    i1 = jnp.min(jnp.where(s2 == m1, iota8, E), axis=1, keepdims=True)
    w0_ref[...] = 1.0 / (1.0 + jnp.exp(m1 - m0))
    w1_ref[...] = 1.0 / (1.0 + jnp.exp(m0 - m1))
    oh0 = (iota8 == i0).astype(jnp.float32)
    oh1 = (iota8 == i1).astype(jnp.float32)
    counts = (jnp.sum(oh0, axis=0, keepdims=True)
              + jnp.sum(oh1, axis=0, keepdims=True))
    nb = jnp.floor((counts + (BR - 1)) * (1.0 / BR))
    ii = lax.broadcasted_iota(jnp.int32, (E, E), 0)
    jj = lax.broadcasted_iota(jnp.int32, (E, E), 1)
    tl = (ii <= jj).astype(jnp.float32)
    cum = lax.dot_general(nb, tl, (((1,), (0,)), ((), ())),
                          preferred_element_type=jnp.float32)
    off = BR * (cum - nb)
    bb = lax.broadcasted_iota(jnp.int32, (NBLK, E), 0).astype(jnp.float32)
    be_ref[...] = jnp.minimum(
        jnp.sum((bb >= cum).astype(jnp.int32), axis=1, keepdims=True), E - 1)
    nv_ref[...] = cum[:, E - 1:E].astype(jnp.int32)
    ci = lax.broadcasted_iota(jnp.int32, (CH, CH), 0)
    cj = lax.broadcasted_iota(jnp.int32, (CH, CH), 1)
    tri = (ci >= cj).astype(jnp.float32)
    m_all = jnp.concatenate([oh0, oh1], axis=0)
    carry = jnp.zeros((1, E), jnp.float32)
    for c in range(NS // CH):
        mc = m_all[c * CH:(c + 1) * CH, :]
        r = lax.dot_general(tri, mc, (((1,), (0,)), ((), ())),
                            preferred_element_type=jnp.float32) + carry
        carry = carry + jnp.sum(mc, axis=0, keepdims=True)
        dest_c = jnp.sum(mc * (off + r - 1.0), axis=1, keepdims=True)
        dest_ref[c * CH:(c + 1) * CH, :] = dest_c.astype(jnp.int32)


def _routing(x2, norm_scale, router_w):
    return pl.pallas_call(
        _routing_body,
        out_shape=(
            jax.ShapeDtypeStruct((NT, DIM), jnp.float32),
            jax.ShapeDtypeStruct((NS, 1), jnp.int32),
            jax.ShapeDtypeStruct((NBLK, 1), jnp.int32),
            jax.ShapeDtypeStruct((1, 1), jnp.int32),
            jax.ShapeDtypeStruct((NT, 1), jnp.float32),
            jax.ShapeDtypeStruct((NT, 1), jnp.float32),
        ),
    )(x2, norm_scale.reshape(1, DIM), router_w)


_NC = 2
_NW = 32
_RXS = NPAD // _NW
_RDG = NS // _NW
_HW = DIM // 2


def _sc_mesh():
    return plsc.VectorSubcoreMesh(core_axis_name="c", subcore_axis_name="s")


def _sc_scatter_xs(dest, xn):
    @functools.partial(
        pl.kernel,
        mesh=_sc_mesh(),
        out_type=jax.ShapeDtypeStruct((NPAD, DIM), jnp.float32),
        scratch_types=[
            pltpu.VMEM((56,), jnp.int32),
            pltpu.VMEM((56,), jnp.int32),
            pltpu.VMEM((16,), jnp.int32),
            pltpu.VMEM((56, DIM), jnp.float32),
            pltpu.VMEM((56, DIM), jnp.float32),
            pltpu.SemaphoreType.DMA,
            pltpu.SemaphoreType.DMA,
            pltpu.SemaphoreType.DMA,
            pltpu.SemaphoreType.DMA,
        ],
    )
    def k(dest_hbm, xn_hbm, xs_hbm, idx0, idx1, idx2, rows0, rows1,
          si0, si1, sr0, sr1):
        wid = lax.axis_index("s") * _NC + lax.axis_index("c")
        base = pl.multiple_of(wid * _RDG, _RDG)
        tok = pl.multiple_of(base & (NT - 1), _RDG)
        ci0 = pltpu.async_copy(dest_hbm.at[pl.ds(base, 56)], idx0, si0)
        ci1 = pltpu.async_copy(dest_hbm.at[pl.ds(base + 56, 56)], idx1, si1)
        ci2 = pltpu.async_copy(dest_hbm.at[pl.ds(base + 112, 16)], idx2, sr1)
        cr0 = pltpu.async_copy(xn_hbm.at[pl.ds(tok, 56)], rows0, sr0)
        cr1 = pltpu.async_copy(xn_hbm.at[pl.ds(tok + 56, 56)], rows1, sr0)
        ci0.wait()
        cr0.wait()
        w0 = pltpu.async_copy(rows0, xs_hbm.at[idx0], si0)
        ci1.wait()
        cr1.wait()
        w1 = pltpu.async_copy(rows1, xs_hbm.at[idx1], si1)
        w0.wait()
        cr2 = pltpu.async_copy(xn_hbm.at[pl.ds(tok + 112, 16)],
                               rows0.at[pl.ds(0, 16)], sr0)
        ci2.wait()
        cr2.wait()
        w2 = pltpu.async_copy(rows0.at[pl.ds(0, 16)], xs_hbm.at[idx2], sr1)
        w1.wait()
        w2.wait()

    return k(dest, xn)


def _sc_gather_comb(dest, d_sorted):
    @functools.partial(
        pl.kernel,
        mesh=_sc_mesh(),
        out_type=jax.ShapeDtypeStruct((NS, DIM), jnp.float32),
        scratch_types=[
            pltpu.VMEM((56,), jnp.int32),
            pltpu.VMEM((56,), jnp.int32),
            pltpu.VMEM((16,), jnp.int32),
            pltpu.VMEM((56, DIM), jnp.float32),
            pltpu.VMEM((56, DIM), jnp.float32),
            pltpu.SemaphoreType.DMA,
            pltpu.SemaphoreType.DMA,
            pltpu.SemaphoreType.DMA,
            pltpu.SemaphoreType.DMA,
        ],
    )
    def k(dest_hbm, d_hbm, dg_hbm, idx0, idx1, idx2, rows0, rows1,
          si0, si1, sr0, sr1):
        wid = lax.axis_index("s") * _NC + lax.axis_index("c")
        base = pl.multiple_of(wid * _RDG, _RDG)
        ci0 = pltpu.async_copy(dest_hbm.at[pl.ds(base, 56)], idx0, si0)
        ci1 = pltpu.async_copy(dest_hbm.at[pl.ds(base + 56, 56)], idx1, si1)
        ci2 = pltpu.async_copy(dest_hbm.at[pl.ds(base + 112, 16)], idx2, sr1)
        ci0.wait()
        g0 = pltpu.async_copy(d_hbm.at[idx0], rows0, sr0)
        ci1.wait()
        g1 = pltpu.async_copy(d_hbm.at[idx1], rows1, sr0)
        g0.wait()
        o0 = pltpu.async_copy(rows0, dg_hbm.at[pl.ds(base, 56)], si0)
        g1.wait()
        o1 = pltpu.async_copy(rows1, dg_hbm.at[pl.ds(base + 56, 56)], si1)
        o0.wait()
        ci2.wait()
        g2 = pltpu.async_copy(d_hbm.at[idx2], rows0.at[pl.ds(0, 16)], sr0)
        g2.wait()
        o2 = pltpu.async_copy(rows0.at[pl.ds(0, 16)],
                              dg_hbm.at[pl.ds(base + 112, 16)], sr1)
        o1.wait()
        o2.wait()

    return k(dest, d_sorted)


def _gmm_body(be_ref, nv_ref, xs_ref, uw_ref, gw_ref, dw_ref, out_ref):
    valid = pl.program_id(0) < nv_ref[0]

    @pl.when(valid)
    def _():
        _gmm_block(xs_ref, uw_ref, gw_ref, dw_ref, out_ref)


def _gmm_block(xs_ref, uw_ref, gw_ref, dw_ref, out_ref):
    xb = xs_ref[...].astype(jnp.bfloat16)
    u = lax.dot_general(xb, uw_ref[0, 0], (((1,), (1,)), ((), ())),
                        preferred_element_type=jnp.float32)
    g = lax.dot_general(xb, gw_ref[0, 0], (((1,), (1,)), ((), ())),
                        preferred_element_type=jnp.float32)
    ub = u.astype(jnp.bfloat16)
    gf = g.astype(jnp.bfloat16).astype(jnp.float32)
    sil = (gf / (1.0 + jnp.exp(-gf))).astype(jnp.bfloat16)
    h = ub * sil
    d = lax.dot_general(h, dw_ref[0], (((1,), (1,)), ((), ())),
                        preferred_element_type=jnp.float32)
    out_ref[...] = d.astype(jnp.bfloat16).astype(jnp.float32)


def _gmm(be, nv, xs_bf, up_bf, down_bf):
    grid_spec = pltpu.PrefetchScalarGridSpec(
        num_scalar_prefetch=2,
        grid=(NBLK,),
        in_specs=[
            pl.BlockSpec((BR, DIM), lambda i, be, nv: (i, 0)),
            pl.BlockSpec((1, 1, HIDDEN, DIM), lambda i, be, nv: (be[i], 0, 0, 0)),
            pl.BlockSpec((1, 1, HIDDEN, DIM), lambda i, be, nv: (be[i], 1, 0, 0)),
            pl.BlockSpec((1, DIM, HIDDEN), lambda i, be, nv: (be[i], 0, 0)),
        ],
        out_specs=pl.BlockSpec((BR, DIM), lambda i, be, nv: (i, 0)),
    )
    f = pl.pallas_call(
        _gmm_body,
        grid_spec=grid_spec,
        out_shape=jax.ShapeDtypeStruct((NPAD, DIM), jnp.float32),
    )
    up4 = up_bf.reshape(E, 2, HIDDEN, DIM)
    return f(be, nv, xs_bf, up4, up4, down_bf)


def _combine_body(d0_ref, d1_ref, w0_ref, w1_ref, x_ref, o_ref):
    o_ref[...] = (w0_ref[...] * d0_ref[...] + w1_ref[...] * d1_ref[...]
                  + x_ref[...])


def _combine(dg, w0, w1, x2):
    nb = NT // BR
    return pl.pallas_call(
        _combine_body,
        grid=(nb,),
        in_specs=[
            pl.BlockSpec((BR, DIM), lambda i: (i, 0)),
            pl.BlockSpec((BR, DIM), lambda i: (i + NT // BR, 0)),
            pl.BlockSpec((BR, 1), lambda i: (i, 0)),
            pl.BlockSpec((BR, 1), lambda i: (i, 0)),
            pl.BlockSpec((BR, DIM), lambda i: (i, 0)),
        ],
        out_specs=pl.BlockSpec((BR, DIM), lambda i: (i, 0)),
        out_shape=jax.ShapeDtypeStruct((NT, DIM), jnp.float32),
    )(dg, dg, w0, w1, x2)


def kernel(x, norm_scale, router_w, up_w, down_w):
    x2 = x.reshape(NT, DIM)
    xn_bf, dest2, be2, nv2, w0, w1 = _routing(x2, norm_scale, router_w)
    dest = dest2.reshape(NS)
    be = be2.reshape(NBLK)
    nv = nv2.reshape(1)
    up_bf = up_w.astype(jnp.bfloat16)
    down_bf = down_w.astype(jnp.bfloat16)
    xs_bf = _sc_scatter_xs(dest, xn_bf)
    d_sorted = _gmm(be, nv, xs_bf, up_bf, down_bf)
    dg = _sc_gather_comb(dest, d_sorted)
    out = _combine(dg, w0, w1, x2)
    return out.reshape(x.shape)
